# baseline (speedup 1.0000x reference)
import jax
import jax.numpy as jnp
from jax.experimental import pallas as pl
from jax.experimental.pallas import tpu as pltpu

_BH = 256


def _router_block(x_ref, w1_ref, b1_ref, w2_ref, b2_ref, rw_ref, tw_ref, ti_ref):
    e_dim = rw_ref.shape[-1]
    h_dim = w1_ref.shape[-1]
    x = x_ref[...]

    logits = jnp.zeros((x.shape[0], e_dim), jnp.float32)
    for k in range(0, h_dim, _BH):
        hk = jnp.dot(x, w1_ref[:, k:k + _BH], preferred_element_type=jnp.float32)
        hk = jnp.maximum(hk + b1_ref[:, k:k + _BH], 0.0)
        logits = logits + jnp.dot(hk, w2_ref[k:k + _BH, :],
                                  preferred_element_type=jnp.float32)
    logits = logits + b2_ref[...]

    ids = jax.lax.broadcasted_iota(jnp.int32, logits.shape, 1)
    m1 = jnp.max(logits, axis=1, keepdims=True)
    i1 = jnp.min(jnp.where(logits == m1, ids, e_dim), axis=1, keepdims=True)
    masked = jnp.where(ids == i1, -jnp.inf, logits)
    m2 = jnp.max(masked, axis=1, keepdims=True)
    i2 = jnp.min(jnp.where(masked == m2, ids, e_dim), axis=1, keepdims=True)

    e = jnp.exp(logits - m1)
    z = jnp.sum(e, axis=1, keepdims=True)
    rw_ref[...] = e / z

    w1v = 1.0 / (1.0 + jnp.exp(m2 - m1))
    tw_ref[...] = jnp.concatenate([w1v, 1.0 - w1v], axis=1)
    ti_ref[...] = jnp.concatenate([i1, i2], axis=1)


def kernel(x, W1, b1, W2, b2, inference_state):
    del inference_state
    t, d = x.shape
    h_dim = W1.shape[1]
    e_dim = W2.shape[1]
    bt = min(512, t)

    rw, tw, ti = pl.pallas_call(
        _router_block,
        grid=(t // bt,),
        in_specs=[
            pl.BlockSpec((bt, d), lambda i: (i, 0)),
            pl.BlockSpec((d, h_dim), lambda i: (0, 0)),
            pl.BlockSpec((1, h_dim), lambda i: (0, 0)),
            pl.BlockSpec((h_dim, e_dim), lambda i: (0, 0)),
            pl.BlockSpec((1, e_dim), lambda i: (0, 0)),
        ],
        out_specs=[
            pl.BlockSpec((bt, e_dim), lambda i: (i, 0)),
            pl.BlockSpec((bt, 2), lambda i: (i, 0)),
            pl.BlockSpec((bt, 2), lambda i: (i, 0)),
        ],
        out_shape=[
            jax.ShapeDtypeStruct((t, e_dim), jnp.float32),
            jax.ShapeDtypeStruct((t, 2), jnp.float32),
            jax.ShapeDtypeStruct((t, 2), jnp.int32),
        ],
        compiler_params=pltpu.CompilerParams(
            dimension_semantics=("arbitrary",),
            vmem_limit_bytes=60 * 1024 * 1024,
        ),
    )(x, W1, b1.reshape(1, h_dim), W2, b2.reshape(1, e_dim))
    return (tw, rw, ti)


# speedup vs baseline: 1.6225x; 1.6225x over previous
import jax
import jax.numpy as jnp
from jax.experimental import pallas as pl
from jax.experimental.pallas import tpu as pltpu


def _router_block(x_ref, w1_ref, b1_ref, w2_ref, b2_ref,
                  rw_ref, tw_ref, ti_ref, h_ref):
    i = pl.program_id(0)
    n = pl.num_programs(0)
    e_dim = rw_ref.shape[-1]

    @pl.when(i < n - 1)
    def _stage1():
        h = jnp.dot(x_ref[...], w1_ref[...], preferred_element_type=jnp.float32)
        h_ref[jax.lax.rem(i, 2)] = jnp.maximum(h + b1_ref[...], 0.0)

    @pl.when(i > 0)
    def _stage2():
        h = h_ref[jax.lax.rem(i + 1, 2)]
        logits = jnp.dot(h, w2_ref[...], preferred_element_type=jnp.float32)
        logits = logits + b2_ref[...]

        ids = jax.lax.broadcasted_iota(jnp.int32, logits.shape, 1)
        m1 = jnp.max(logits, axis=1, keepdims=True)
        i1 = jnp.min(jnp.where(logits == m1, ids, e_dim), axis=1, keepdims=True)
        masked = jnp.where(ids == i1, -jnp.inf, logits)
        m2 = jnp.max(masked, axis=1, keepdims=True)
        i2 = jnp.min(jnp.where(masked == m2, ids, e_dim), axis=1, keepdims=True)

        e = jnp.exp(logits - m1)
        z = jnp.sum(e, axis=1, keepdims=True)
        rw_ref[...] = e / z

        w1v = 1.0 / (1.0 + jnp.exp(m2 - m1))
        tw_ref[...] = jnp.concatenate([w1v, 1.0 - w1v], axis=1)
        ti_ref[...] = jnp.concatenate([i1, i2], axis=1)


def kernel(x, W1, b1, W2, b2, inference_state):
    del inference_state
    t, d = x.shape
    h_dim = W1.shape[1]
    e_dim = W2.shape[1]
    bt = min(512, t)
    nblk = t // bt

    def _prev(i):
        return jnp.maximum(i - 1, 0)

    rw, tw, ti = pl.pallas_call(
        _router_block,
        grid=(nblk + 1,),
        in_specs=[
            pl.BlockSpec((bt, d), lambda i: (jnp.minimum(i, nblk - 1), 0)),
            pl.BlockSpec((d, h_dim), lambda i: (0, 0)),
            pl.BlockSpec((1, h_dim), lambda i: (0, 0)),
            pl.BlockSpec((h_dim, e_dim), lambda i: (0, 0)),
            pl.BlockSpec((1, e_dim), lambda i: (0, 0)),
        ],
        out_specs=[
            pl.BlockSpec((bt, e_dim), lambda i: (_prev(i), 0)),
            pl.BlockSpec((bt, 2), lambda i: (_prev(i), 0)),
            pl.BlockSpec((bt, 2), lambda i: (_prev(i), 0)),
        ],
        out_shape=[
            jax.ShapeDtypeStruct((t, e_dim), jnp.float32),
            jax.ShapeDtypeStruct((t, 2), jnp.float32),
            jax.ShapeDtypeStruct((t, 2), jnp.int32),
        ],
        scratch_shapes=[pltpu.VMEM((2, bt, h_dim), jnp.float32)],
        compiler_params=pltpu.CompilerParams(
            dimension_semantics=("arbitrary",),
            vmem_limit_bytes=60 * 1024 * 1024,
        ),
    )(x, W1, b1.reshape(1, h_dim), W2, b2.reshape(1, e_dim))
    return (tw, rw, ti)
